# baseline (speedup 1.0000x reference)
import functools

import jax
import jax.numpy as jnp
from jax import lax
from jax.experimental import pallas as pl
from jax.experimental.pallas import tpu as pltpu
from jax.experimental.pallas import tpu_sc as plsc

N_NODES = 10000
N_HALF = 5000
N_PAD = 5120
COORD_DIM = 64
FEAT_DIM = 256
M = 16

_NC = 2
_NS = 16
_NW = _NC * _NS
_ROWS_PER_W = N_PAD // _NW


def _sc_gather_feats(h):
    mesh = plsc.VectorSubcoreMesh(core_axis_name="c", subcore_axis_name="s")

    @functools.partial(
        pl.kernel,
        mesh=mesh,
        out_type=jax.ShapeDtypeStruct((N_PAD, FEAT_DIM), jnp.float32),
        scratch_types=[
            pltpu.VMEM((_ROWS_PER_W,), jnp.int32),
            pltpu.VMEM((_ROWS_PER_W, FEAT_DIM), jnp.float32),
            pltpu.SemaphoreType.DMA,
        ],
    )
    def k(h_hbm, out_hbm, idx_v, buf, sem):
        wid = lax.axis_index("s") * _NC + lax.axis_index("c")
        base = wid * _ROWS_PER_W
        for c in range(_ROWS_PER_W // 16):
            lane = lax.iota(jnp.int32, 16)
            ev = jnp.minimum((base + c * 16 + lane) * 2, N_NODES - 2)
            idx_v[pl.ds(c * 16, 16)] = ev
        pltpu.async_copy(h_hbm.at[idx_v], buf, sem).wait()
        pltpu.sync_copy(buf, out_hbm.at[pl.ds(base, _ROWS_PER_W)])

    return k(h)


_RB = 256
_KT = 512
_NKT = N_PAD // _KT
_CHUNK = 8


def _knn_body(s2_ref, j_ref, i_ref, sh_ref, d2_ref):
    b = pl.program_id(0)
    blk = s2_ref[pl.ds(b * _RB, _RB), :]
    q = blk[:, COORD_DIM:]
    sh_ref[...] = blk[:, :COORD_DIM]
    qs = jnp.sum(q * q, axis=1, keepdims=True)
    for kt in range(_NKT):
        keys_t = s2_ref[pl.ds(kt * _KT, _KT), :COORD_DIM]
        dp = lax.dot_general(
            q, keys_t, dimension_numbers=(((1,), (1,)), ((), ())),
            preferred_element_type=jnp.float32,
        )
        ks_col = jnp.sum(keys_t * keys_t, axis=1, keepdims=True)
        ks_row = lax.transpose(ks_col, (1, 0))
        d2 = qs - 2.0 * dp + ks_row
        if (kt + 1) * _KT > N_HALF:
            col = kt * _KT + lax.broadcasted_iota(jnp.int32, (_RB, _KT), 1)
            d2 = jnp.where(col >= N_HALF, jnp.inf, d2)
        d2_ref[:, pl.ds(kt * _KT, _KT)] = d2

    def chunk(c, carry):
        rows = pl.ds(c * _CHUNK, _CHUNK)
        lane = lax.broadcasted_iota(jnp.int32, (_CHUNK, 128), 1)
        big = jnp.int32(1 << 30)
        inf = jnp.float32(jnp.inf)
        v1 = jnp.full((_CHUNK, 128), inf)
        v2 = jnp.full((_CHUNK, 128), inf)
        v3 = jnp.full((_CHUNK, 128), inf)
        i1 = jnp.full((_CHUNK, 128), big)
        i2 = jnp.full((_CHUNK, 128), big)
        i3 = jnp.full((_CHUNK, 128), big)
        for t in range(N_PAD // 128):
            x = d2_ref[rows, pl.ds(t * 128, 128)]
            ix = lane + t * 128
            c1 = x < v1
            c2 = x < v2
            c3 = x < v3
            v3 = jnp.where(c2, v2, jnp.where(c3, x, v3))
            i3 = jnp.where(c2, i2, jnp.where(c3, ix, i3))
            v2 = jnp.where(c1, v1, jnp.where(c2, x, v2))
            i2 = jnp.where(c1, i1, jnp.where(c2, ix, i2))
            v1 = jnp.where(c1, x, v1)
            i1 = jnp.where(c1, ix, i1)
        cols = []
        vmin = v1
        am = i1
        for m in range(M):
            vmin = v1
            am = i1
            for sh in (64, 32, 16, 8, 4, 2, 1):
                rv = pltpu.roll(vmin, sh, 1)
                ri = pltpu.roll(am, sh, 1)
                cl = (rv < vmin) | ((rv == vmin) & (ri < am))
                vmin = jnp.where(cl, rv, vmin)
                am = jnp.where(cl, ri, am)
            cols.append(am[:, :1])
            if m < M - 1:
                win = (v1 == vmin) & (i1 == am)
                v1 = jnp.where(win, v2, v1)
                i1 = jnp.where(win, i2, i1)
                v2 = jnp.where(win, v3, v2)
                i2 = jnp.where(win, i3, i2)
                v3 = jnp.where(win, inf, v3)
        nbr = jnp.concatenate(cols, axis=1)
        j_ref[rows, :] = nbr * 2
        acc = jnp.zeros((_CHUNK, 128), jnp.int32)
        for t in range(N_PAD // 128):
            x = d2_ref[rows, pl.ds(t * 128, 128)]
            ix = lane + t * 128
            le = (x < vmin) | ((x == vmin) & (ix <= am))
            acc = acc + le.astype(jnp.int32)
        cnt = jnp.sum(acc, axis=1, keepdims=True)
        bad = jnp.max(jnp.abs(cnt - M)) > 0

        @pl.when(bad)
        def _fallback():
            d = d2_ref[rows, :]
            ci = lax.broadcasted_iota(jnp.int32, (_CHUNK, N_PAD), 1)
            fcols = []
            for m in range(M):
                fvmin = jnp.min(d, axis=1, keepdims=True)
                fam = jnp.min(jnp.where(d == fvmin, ci, N_PAD), axis=1,
                              keepdims=True)
                fcols.append(fam)
                if m < M - 1:
                    d = jnp.where(ci == fam, inf, d)
            j_ref[rows, :] = jnp.concatenate(fcols, axis=1) * 2

        return carry

    lax.fori_loop(0, _RB // _CHUNK, chunk, 0)
    rowid = lax.broadcasted_iota(jnp.int32, (_RB, M), 0)
    i_ref[...] = 2 * (b * _RB + rowid) + 1


def _tc_knn(s2p):
    grid = N_PAD // _RB
    return pl.pallas_call(
        _knn_body,
        grid=(grid,),
        in_specs=[pl.BlockSpec((N_PAD, 2 * COORD_DIM), lambda i: (0, 0))],
        out_specs=[
            pl.BlockSpec((_RB, M), lambda i: (i, 0)),
            pl.BlockSpec((_RB, M), lambda i: (i, 0)),
            pl.BlockSpec((_RB, COORD_DIM), lambda i: (i, 0)),
        ],
        out_shape=[
            jax.ShapeDtypeStruct((N_PAD, M), jnp.int32),
            jax.ShapeDtypeStruct((N_PAD, M), jnp.int32),
            jax.ShapeDtypeStruct((N_PAD, COORD_DIM), jnp.float32),
        ],
        scratch_shapes=[pltpu.VMEM((_RB, N_PAD), jnp.float32)],
    )(s2p)


def kernel(s_l, h):
    h_hor_pad = _sc_gather_feats(h)
    s2 = s_l.reshape(N_HALF, 2 * COORD_DIM)
    s2p = jnp.pad(s2, ((0, N_PAD - N_HALF), (0, 0)))
    j_pad, i_pad, s_hor_pad = _tc_knn(s2p)
    i = i_pad[:N_HALF].reshape(-1)
    j = j_pad[:N_HALF].reshape(-1)
    return (i, j, h_hor_pad[:N_HALF], s_hor_pad[:N_HALF])


# speedup vs baseline: 1.9983x; 1.9983x over previous
import functools

import jax
import jax.numpy as jnp
from jax import lax
from jax.experimental import pallas as pl
from jax.experimental.pallas import tpu as pltpu
from jax.experimental.pallas import tpu_sc as plsc

N_NODES = 10000
N_HALF = 5000
N_PAD = 5120
COORD_DIM = 64
FEAT_DIM = 256
M = 16

_NC = 2
_NS = 16
_NW = _NC * _NS
_ROWS_PER_W = N_PAD // _NW


def _sc_gather_feats(h):
    mesh = plsc.VectorSubcoreMesh(core_axis_name="c", subcore_axis_name="s")

    @functools.partial(
        pl.kernel,
        mesh=mesh,
        out_type=jax.ShapeDtypeStruct((N_PAD, FEAT_DIM), jnp.float32),
        scratch_types=[
            pltpu.VMEM((_ROWS_PER_W,), jnp.int32),
            pltpu.VMEM((_ROWS_PER_W, FEAT_DIM), jnp.float32),
            pltpu.SemaphoreType.DMA,
        ],
    )
    def k(h_hbm, out_hbm, idx_v, buf, sem):
        wid = lax.axis_index("s") * _NC + lax.axis_index("c")
        base = wid * _ROWS_PER_W
        for c in range(_ROWS_PER_W // 16):
            lane = lax.iota(jnp.int32, 16)
            ev = jnp.minimum((base + c * 16 + lane) * 2, N_NODES - 2)
            idx_v[pl.ds(c * 16, 16)] = ev
        pltpu.async_copy(h_hbm.at[idx_v], buf, sem).wait()
        pltpu.sync_copy(buf, out_hbm.at[pl.ds(base, _ROWS_PER_W)])

    return k(h)


_RB = 256
_KT = 512
_NKT = N_PAD // _KT
_CHUNK = 8


def _knn_body(s2_ref, j_ref, i_ref, sh_ref, d2_ref):
    b = pl.program_id(0)
    blk = s2_ref[pl.ds(b * _RB, _RB), :]
    q = blk[:, COORD_DIM:]
    sh_ref[...] = blk[:, :COORD_DIM]
    qs = jnp.sum(q * q, axis=1, keepdims=True)
    for kt in range(_NKT):
        keys_t = s2_ref[pl.ds(kt * _KT, _KT), :COORD_DIM]
        dp = lax.dot_general(
            q, keys_t, dimension_numbers=(((1,), (1,)), ((), ())),
            preferred_element_type=jnp.float32,
        )
        ks_col = jnp.sum(keys_t * keys_t, axis=1, keepdims=True)
        ks_row = lax.transpose(ks_col, (1, 0))
        d2 = qs - 2.0 * dp + ks_row
        if (kt + 1) * _KT > N_HALF:
            col = kt * _KT + lax.broadcasted_iota(jnp.int32, (_RB, _KT), 1)
            d2 = jnp.where(col >= N_HALF, jnp.inf, d2)
        d2_ref[:, pl.ds(kt * _KT, _KT)] = d2

    def chunk(c, carry):
        rows = pl.ds(c * _CHUNK, _CHUNK)
        lane = lax.broadcasted_iota(jnp.int32, (_CHUNK, 128), 1)
        big = jnp.int32(1 << 30)
        inf = jnp.float32(jnp.inf)
        v1 = jnp.full((_CHUNK, 128), inf)
        v2 = jnp.full((_CHUNK, 128), inf)
        v3 = jnp.full((_CHUNK, 128), inf)
        i1 = jnp.full((_CHUNK, 128), big)
        i2 = jnp.full((_CHUNK, 128), big)
        i3 = jnp.full((_CHUNK, 128), big)
        for t in range(N_PAD // 128):
            x = d2_ref[rows, pl.ds(t * 128, 128)]
            ix = lane + t * 128
            c1 = x < v1
            c2 = x < v2
            c3 = x < v3
            v3 = jnp.where(c2, v2, jnp.where(c3, x, v3))
            i3 = jnp.where(c2, i2, jnp.where(c3, ix, i3))
            v2 = jnp.where(c1, v1, jnp.where(c2, x, v2))
            i2 = jnp.where(c1, i1, jnp.where(c2, ix, i2))
            v1 = jnp.where(c1, x, v1)
            i1 = jnp.where(c1, ix, i1)
        cols = []
        vmin = v1
        am = i1
        for m in range(M):
            vmin = jnp.min(v1, axis=1, keepdims=True)
            eqm = v1 == vmin
            am = jnp.min(jnp.where(eqm, i1, big), axis=1, keepdims=True)
            cols.append(am)
            if m < M - 1:
                win = eqm & (i1 == am)
                v1 = jnp.where(win, v2, v1)
                i1 = jnp.where(win, i2, i1)
                v2 = jnp.where(win, v3, v2)
                i2 = jnp.where(win, i3, i2)
                v3 = jnp.where(win, inf, v3)
        nbr = jnp.concatenate(cols, axis=1)
        j_ref[rows, :] = nbr * 2
        acc = jnp.zeros((_CHUNK, 128), jnp.int32)
        for t in range(N_PAD // 128):
            x = d2_ref[rows, pl.ds(t * 128, 128)]
            ix = lane + t * 128
            le = (x < vmin) | ((x == vmin) & (ix <= am))
            acc = acc + le.astype(jnp.int32)
        cnt = jnp.sum(acc, axis=1, keepdims=True)
        bad = jnp.max(jnp.abs(cnt - M)) > 0

        @pl.when(bad)
        def _fallback():
            d = d2_ref[rows, :]
            ci = lax.broadcasted_iota(jnp.int32, (_CHUNK, N_PAD), 1)
            fcols = []
            for m in range(M):
                fvmin = jnp.min(d, axis=1, keepdims=True)
                fam = jnp.min(jnp.where(d == fvmin, ci, N_PAD), axis=1,
                              keepdims=True)
                fcols.append(fam)
                if m < M - 1:
                    d = jnp.where(ci == fam, inf, d)
            j_ref[rows, :] = jnp.concatenate(fcols, axis=1) * 2

        return carry

    lax.fori_loop(0, _RB // _CHUNK, chunk, 0)
    rowid = lax.broadcasted_iota(jnp.int32, (_RB, M), 0)
    i_ref[...] = 2 * (b * _RB + rowid) + 1


def _tc_knn(s2p):
    grid = N_PAD // _RB
    return pl.pallas_call(
        _knn_body,
        grid=(grid,),
        in_specs=[pl.BlockSpec((N_PAD, 2 * COORD_DIM), lambda i: (0, 0))],
        out_specs=[
            pl.BlockSpec((_RB, M), lambda i: (i, 0)),
            pl.BlockSpec((_RB, M), lambda i: (i, 0)),
            pl.BlockSpec((_RB, COORD_DIM), lambda i: (i, 0)),
        ],
        out_shape=[
            jax.ShapeDtypeStruct((N_PAD, M), jnp.int32),
            jax.ShapeDtypeStruct((N_PAD, M), jnp.int32),
            jax.ShapeDtypeStruct((N_PAD, COORD_DIM), jnp.float32),
        ],
        scratch_shapes=[pltpu.VMEM((_RB, N_PAD), jnp.float32)],
    )(s2p)


def kernel(s_l, h):
    h_hor_pad = _sc_gather_feats(h)
    s2 = s_l.reshape(N_HALF, 2 * COORD_DIM)
    s2p = jnp.pad(s2, ((0, N_PAD - N_HALF), (0, 0)))
    j_pad, i_pad, s_hor_pad = _tc_knn(s2p)
    i = i_pad[:N_HALF].reshape(-1)
    j = j_pad[:N_HALF].reshape(-1)
    return (i, j, h_hor_pad[:N_HALF], s_hor_pad[:N_HALF])


# speedup vs baseline: 6.6406x; 3.3232x over previous
import functools

import jax
import jax.numpy as jnp
from jax import lax
from jax.experimental import pallas as pl
from jax.experimental.pallas import tpu as pltpu
from jax.experimental.pallas import tpu_sc as plsc

N_NODES = 10000
N_HALF = 5000
N_PAD = 5120
COORD_DIM = 64
FEAT_DIM = 256
M = 16

_NC = 2
_NS = 16
_NW = _NC * _NS
_ROWS_PER_W = N_PAD // _NW


def _sc_gather_feats(h):
    mesh = plsc.VectorSubcoreMesh(core_axis_name="c", subcore_axis_name="s")

    @functools.partial(
        pl.kernel,
        mesh=mesh,
        out_type=jax.ShapeDtypeStruct((N_PAD, FEAT_DIM), jnp.float32),
        scratch_types=[
            pltpu.VMEM((_ROWS_PER_W,), jnp.int32),
            pltpu.VMEM((_ROWS_PER_W, FEAT_DIM), jnp.float32),
            pltpu.SemaphoreType.DMA,
        ],
    )
    def k(h_hbm, out_hbm, idx_v, buf, sem):
        wid = lax.axis_index("s") * _NC + lax.axis_index("c")
        base = wid * _ROWS_PER_W
        for c in range(_ROWS_PER_W // 16):
            lane = lax.iota(jnp.int32, 16)
            ev = jnp.minimum((base + c * 16 + lane) * 2, N_NODES - 2)
            idx_v[pl.ds(c * 16, 16)] = ev
        pltpu.async_copy(h_hbm.at[idx_v], buf, sem).wait()
        pltpu.sync_copy(buf, out_hbm.at[pl.ds(base, _ROWS_PER_W)])

    return k(h)


_RB = 256
_KT = 512
_NKT = N_PAD // _KT
_CHUNK = 32


def _knn_body(s2_ref, j_ref, i_ref, sh_ref, d2_ref):
    b = pl.program_id(0)
    blk = s2_ref[pl.ds(b * _RB, _RB), :]
    q = blk[:, COORD_DIM:]
    sh_ref[...] = blk[:, :COORD_DIM]
    qs = jnp.sum(q * q, axis=1, keepdims=True)
    for kt in range(_NKT):
        keys_t = s2_ref[pl.ds(kt * _KT, _KT), :COORD_DIM]
        dp = lax.dot_general(
            q, keys_t, dimension_numbers=(((1,), (1,)), ((), ())),
            preferred_element_type=jnp.float32,
        )
        ks_col = jnp.sum(keys_t * keys_t, axis=1, keepdims=True)
        ks_row = lax.transpose(ks_col, (1, 0))
        d2 = qs - 2.0 * dp + ks_row
        if (kt + 1) * _KT > N_HALF:
            col = kt * _KT + lax.broadcasted_iota(jnp.int32, (_RB, _KT), 1)
            d2 = jnp.where(col >= N_HALF, jnp.inf, d2)
        d2_ref[:, pl.ds(kt * _KT, _KT)] = d2

    def chunk(c, carry):
        rows = pl.ds(c * _CHUNK, _CHUNK)
        lane = lax.broadcasted_iota(jnp.int32, (_CHUNK, 128), 1)
        big = jnp.int32(1 << 30)
        inf = jnp.float32(jnp.inf)
        v1 = jnp.full((_CHUNK, 128), inf)
        v2 = jnp.full((_CHUNK, 128), inf)
        v3 = jnp.full((_CHUNK, 128), inf)
        i1 = jnp.full((_CHUNK, 128), big)
        i2 = jnp.full((_CHUNK, 128), big)
        i3 = jnp.full((_CHUNK, 128), big)
        for t in range(N_PAD // 128):
            x = d2_ref[rows, pl.ds(t * 128, 128)]
            ix = lane + t * 128
            c1 = x < v1
            c2 = x < v2
            c3 = x < v3
            v3 = jnp.where(c2, v2, jnp.where(c3, x, v3))
            i3 = jnp.where(c2, i2, jnp.where(c3, ix, i3))
            v2 = jnp.where(c1, v1, jnp.where(c2, x, v2))
            i2 = jnp.where(c1, i1, jnp.where(c2, ix, i2))
            v1 = jnp.where(c1, x, v1)
            i1 = jnp.where(c1, ix, i1)
        cols = []
        vmin = v1
        am = i1
        for m in range(M):
            vmin = jnp.min(v1, axis=1, keepdims=True)
            eqm = v1 == vmin
            am = jnp.min(jnp.where(eqm, i1, big), axis=1, keepdims=True)
            cols.append(am)
            if m < M - 1:
                win = eqm & (i1 == am)
                v1 = jnp.where(win, v2, v1)
                i1 = jnp.where(win, i2, i1)
                v2 = jnp.where(win, v3, v2)
                i2 = jnp.where(win, i3, i2)
                v3 = jnp.where(win, inf, v3)
        nbr = jnp.concatenate(cols, axis=1)
        j_ref[rows, :] = nbr * 2
        acc = jnp.zeros((_CHUNK, 128), jnp.int32)
        for t in range(N_PAD // 128):
            x = d2_ref[rows, pl.ds(t * 128, 128)]
            ix = lane + t * 128
            le = (x < vmin) | ((x == vmin) & (ix <= am))
            acc = acc + le.astype(jnp.int32)
        cnt = jnp.sum(acc, axis=1, keepdims=True)
        bad = jnp.max(jnp.abs(cnt - M)) > 0

        @pl.when(bad)
        def _fallback():
            d = d2_ref[rows, :]
            ci = lax.broadcasted_iota(jnp.int32, (_CHUNK, N_PAD), 1)
            fcols = []
            for m in range(M):
                fvmin = jnp.min(d, axis=1, keepdims=True)
                fam = jnp.min(jnp.where(d == fvmin, ci, N_PAD), axis=1,
                              keepdims=True)
                fcols.append(fam)
                if m < M - 1:
                    d = jnp.where(ci == fam, inf, d)
            j_ref[rows, :] = jnp.concatenate(fcols, axis=1) * 2

        return carry

    lax.fori_loop(0, _RB // _CHUNK, chunk, 0)
    rowid = lax.broadcasted_iota(jnp.int32, (_RB, M), 0)
    i_ref[...] = 2 * (b * _RB + rowid) + 1


def _tc_knn(s2p):
    grid = N_PAD // _RB
    return pl.pallas_call(
        _knn_body,
        grid=(grid,),
        in_specs=[pl.BlockSpec((N_PAD, 2 * COORD_DIM), lambda i: (0, 0))],
        out_specs=[
            pl.BlockSpec((_RB, M), lambda i: (i, 0)),
            pl.BlockSpec((_RB, M), lambda i: (i, 0)),
            pl.BlockSpec((_RB, COORD_DIM), lambda i: (i, 0)),
        ],
        out_shape=[
            jax.ShapeDtypeStruct((N_PAD, M), jnp.int32),
            jax.ShapeDtypeStruct((N_PAD, M), jnp.int32),
            jax.ShapeDtypeStruct((N_PAD, COORD_DIM), jnp.float32),
        ],
        scratch_shapes=[pltpu.VMEM((_RB, N_PAD), jnp.float32)],
    )(s2p)


def kernel(s_l, h):
    h_hor_pad = _sc_gather_feats(h)
    s2 = s_l.reshape(N_HALF, 2 * COORD_DIM)
    s2p = jnp.pad(s2, ((0, N_PAD - N_HALF), (0, 0)))
    j_pad, i_pad, s_hor_pad = _tc_knn(s2p)
    i = i_pad[:N_HALF].reshape(-1)
    j = j_pad[:N_HALF].reshape(-1)
    return (i, j, h_hor_pad[:N_HALF], s_hor_pad[:N_HALF])


# speedup vs baseline: 10.7370x; 1.6169x over previous
import functools

import jax
import jax.numpy as jnp
from jax import lax
from jax.experimental import pallas as pl
from jax.experimental.pallas import tpu as pltpu
from jax.experimental.pallas import tpu_sc as plsc

N_NODES = 10000
N_HALF = 5000
N_PAD = 5120
COORD_DIM = 64
FEAT_DIM = 256
M = 16

_NC = 2
_NS = 16
_NW = _NC * _NS
_ROWS_PER_W = N_PAD // _NW


def _sc_gather_feats(h):
    mesh = plsc.VectorSubcoreMesh(core_axis_name="c", subcore_axis_name="s")

    @functools.partial(
        pl.kernel,
        mesh=mesh,
        out_type=jax.ShapeDtypeStruct((N_PAD, FEAT_DIM), jnp.float32),
        scratch_types=[
            pltpu.VMEM((_ROWS_PER_W,), jnp.int32),
            pltpu.VMEM((_ROWS_PER_W, FEAT_DIM), jnp.float32),
            pltpu.SemaphoreType.DMA,
        ],
    )
    def k(h_hbm, out_hbm, idx_v, buf, sem):
        wid = lax.axis_index("s") * _NC + lax.axis_index("c")
        base = wid * _ROWS_PER_W
        for c in range(_ROWS_PER_W // 16):
            lane = lax.iota(jnp.int32, 16)
            ev = jnp.minimum((base + c * 16 + lane) * 2, N_NODES - 2)
            idx_v[pl.ds(c * 16, 16)] = ev
        pltpu.async_copy(h_hbm.at[idx_v], buf, sem).wait()
        pltpu.sync_copy(buf, out_hbm.at[pl.ds(base, _ROWS_PER_W)])

    return k(h)


_RB = 256
_KT = 512
_NKT = N_PAD // _KT
_CHUNK = 64


def _knn_body(s2_ref, j_ref, i_ref, sh_ref, d2_ref):
    b = pl.program_id(0)
    blk = s2_ref[pl.ds(b * _RB, _RB), :]
    q = blk[:, COORD_DIM:]
    sh_ref[...] = blk[:, :COORD_DIM]
    qs = jnp.sum(q * q, axis=1, keepdims=True)
    for kt in range(_NKT):
        keys_t = s2_ref[pl.ds(kt * _KT, _KT), :COORD_DIM]
        dp = lax.dot_general(
            q, keys_t, dimension_numbers=(((1,), (1,)), ((), ())),
            preferred_element_type=jnp.float32,
        )
        ks_col = jnp.sum(keys_t * keys_t, axis=1, keepdims=True)
        ks_row = lax.transpose(ks_col, (1, 0))
        d2 = qs - 2.0 * dp + ks_row
        if (kt + 1) * _KT > N_HALF:
            col = kt * _KT + lax.broadcasted_iota(jnp.int32, (_RB, _KT), 1)
            d2 = jnp.where(col >= N_HALF, jnp.inf, d2)
        d2_ref[:, pl.ds(kt * _KT, _KT)] = d2

    def chunk(c, carry):
        rows = pl.ds(c * _CHUNK, _CHUNK)
        lane = lax.broadcasted_iota(jnp.int32, (_CHUNK, 128), 1)
        big = jnp.int32(1 << 30)
        inf = jnp.float32(jnp.inf)
        v1 = jnp.full((_CHUNK, 128), inf)
        v2 = jnp.full((_CHUNK, 128), inf)
        v3 = jnp.full((_CHUNK, 128), inf)
        i1 = jnp.full((_CHUNK, 128), big)
        i2 = jnp.full((_CHUNK, 128), big)
        i3 = jnp.full((_CHUNK, 128), big)
        for t in range(N_PAD // 128):
            x = d2_ref[rows, pl.ds(t * 128, 128)]
            ix = lane + t * 128
            c1 = x < v1
            c2 = x < v2
            c3 = x < v3
            v3 = jnp.where(c2, v2, jnp.where(c3, x, v3))
            i3 = jnp.where(c2, i2, jnp.where(c3, ix, i3))
            v2 = jnp.where(c1, v1, jnp.where(c2, x, v2))
            i2 = jnp.where(c1, i1, jnp.where(c2, ix, i2))
            v1 = jnp.where(c1, x, v1)
            i1 = jnp.where(c1, ix, i1)
        cols = []
        vmin = v1
        am = i1
        for m in range(M):
            vmin = jnp.min(v1, axis=1, keepdims=True)
            eqm = v1 == vmin
            am = jnp.min(jnp.where(eqm, i1, big), axis=1, keepdims=True)
            cols.append(am)
            if m < M - 1:
                win = eqm & (i1 == am)
                v1 = jnp.where(win, v2, v1)
                i1 = jnp.where(win, i2, i1)
                v2 = jnp.where(win, v3, v2)
                i2 = jnp.where(win, i3, i2)
                v3 = jnp.where(win, inf, v3)
        nbr = jnp.concatenate(cols, axis=1)
        j_ref[rows, :] = nbr * 2
        acc = jnp.zeros((_CHUNK, 128), jnp.int32)
        for t in range(N_PAD // 128):
            x = d2_ref[rows, pl.ds(t * 128, 128)]
            ix = lane + t * 128
            le = (x < vmin) | ((x == vmin) & (ix <= am))
            acc = acc + le.astype(jnp.int32)
        cnt = jnp.sum(acc, axis=1, keepdims=True)
        bad = jnp.max(jnp.abs(cnt - M)) > 0

        @pl.when(bad)
        def _fallback():
            d = d2_ref[rows, :]
            ci = lax.broadcasted_iota(jnp.int32, (_CHUNK, N_PAD), 1)
            fcols = []
            for m in range(M):
                fvmin = jnp.min(d, axis=1, keepdims=True)
                fam = jnp.min(jnp.where(d == fvmin, ci, N_PAD), axis=1,
                              keepdims=True)
                fcols.append(fam)
                if m < M - 1:
                    d = jnp.where(ci == fam, inf, d)
            j_ref[rows, :] = jnp.concatenate(fcols, axis=1) * 2

        return carry

    lax.fori_loop(0, _RB // _CHUNK, chunk, 0)
    rowid = lax.broadcasted_iota(jnp.int32, (_RB, M), 0)
    i_ref[...] = 2 * (b * _RB + rowid) + 1


def _tc_knn(s2p):
    grid = N_PAD // _RB
    return pl.pallas_call(
        _knn_body,
        grid=(grid,),
        in_specs=[pl.BlockSpec((N_PAD, 2 * COORD_DIM), lambda i: (0, 0))],
        out_specs=[
            pl.BlockSpec((_RB, M), lambda i: (i, 0)),
            pl.BlockSpec((_RB, M), lambda i: (i, 0)),
            pl.BlockSpec((_RB, COORD_DIM), lambda i: (i, 0)),
        ],
        out_shape=[
            jax.ShapeDtypeStruct((N_PAD, M), jnp.int32),
            jax.ShapeDtypeStruct((N_PAD, M), jnp.int32),
            jax.ShapeDtypeStruct((N_PAD, COORD_DIM), jnp.float32),
        ],
        scratch_shapes=[pltpu.VMEM((_RB, N_PAD), jnp.float32)],
    )(s2p)


def kernel(s_l, h):
    h_hor_pad = _sc_gather_feats(h)
    s2 = s_l.reshape(N_HALF, 2 * COORD_DIM)
    s2p = jnp.pad(s2, ((0, N_PAD - N_HALF), (0, 0)))
    j_pad, i_pad, s_hor_pad = _tc_knn(s2p)
    i = i_pad[:N_HALF].reshape(-1)
    j = j_pad[:N_HALF].reshape(-1)
    return (i, j, h_hor_pad[:N_HALF], s_hor_pad[:N_HALF])


# speedup vs baseline: 14.6366x; 1.3632x over previous
import functools

import jax
import jax.numpy as jnp
from jax import lax
from jax.experimental import pallas as pl
from jax.experimental.pallas import tpu as pltpu
from jax.experimental.pallas import tpu_sc as plsc

N_NODES = 10000
N_HALF = 5000
N_PAD = 5120
COORD_DIM = 64
FEAT_DIM = 256
M = 16

_NC = 2
_NS = 16
_NW = _NC * _NS
_ROWS_PER_W = N_PAD // _NW


def _sc_gather_feats(h):
    mesh = plsc.VectorSubcoreMesh(core_axis_name="c", subcore_axis_name="s")

    @functools.partial(
        pl.kernel,
        mesh=mesh,
        out_type=jax.ShapeDtypeStruct((N_PAD, FEAT_DIM), jnp.float32),
        scratch_types=[
            pltpu.VMEM((_ROWS_PER_W,), jnp.int32),
            pltpu.VMEM((_ROWS_PER_W, FEAT_DIM), jnp.float32),
            pltpu.SemaphoreType.DMA,
        ],
    )
    def k(h_hbm, out_hbm, idx_v, buf, sem):
        wid = lax.axis_index("s") * _NC + lax.axis_index("c")
        base = wid * _ROWS_PER_W
        for c in range(_ROWS_PER_W // 16):
            lane = lax.iota(jnp.int32, 16)
            ev = jnp.minimum((base + c * 16 + lane) * 2, N_NODES - 2)
            idx_v[pl.ds(c * 16, 16)] = ev
        pltpu.async_copy(h_hbm.at[idx_v], buf, sem).wait()
        pltpu.sync_copy(buf, out_hbm.at[pl.ds(base, _ROWS_PER_W)])

    return k(h)


_RB = 256
_KT = 512
_NKT = N_PAD // _KT
_CHUNK = 128


def _knn_body(s2_ref, j_ref, i_ref, sh_ref, d2_ref):
    b = pl.program_id(0)
    blk = s2_ref[pl.ds(b * _RB, _RB), :]
    q = blk[:, COORD_DIM:]
    sh_ref[...] = blk[:, :COORD_DIM]
    qs = jnp.sum(q * q, axis=1, keepdims=True)
    for kt in range(_NKT):
        keys_t = s2_ref[pl.ds(kt * _KT, _KT), :COORD_DIM]
        dp = lax.dot_general(
            q, keys_t, dimension_numbers=(((1,), (1,)), ((), ())),
            preferred_element_type=jnp.float32,
        )
        ks_col = jnp.sum(keys_t * keys_t, axis=1, keepdims=True)
        ks_row = lax.transpose(ks_col, (1, 0))
        d2 = qs - 2.0 * dp + ks_row
        if (kt + 1) * _KT > N_HALF:
            col = kt * _KT + lax.broadcasted_iota(jnp.int32, (_RB, _KT), 1)
            d2 = jnp.where(col >= N_HALF, jnp.inf, d2)
        d2_ref[:, pl.ds(kt * _KT, _KT)] = d2

    def chunk(c, carry):
        rows = pl.ds(c * _CHUNK, _CHUNK)
        lane = lax.broadcasted_iota(jnp.int32, (_CHUNK, 128), 1)
        big = jnp.int32(1 << 30)
        inf = jnp.float32(jnp.inf)
        v1 = jnp.full((_CHUNK, 128), inf)
        v2 = jnp.full((_CHUNK, 128), inf)
        v3 = jnp.full((_CHUNK, 128), inf)
        i1 = jnp.full((_CHUNK, 128), big)
        i2 = jnp.full((_CHUNK, 128), big)
        i3 = jnp.full((_CHUNK, 128), big)
        for t in range(N_PAD // 128):
            x = d2_ref[rows, pl.ds(t * 128, 128)]
            ix = lane + t * 128
            c1 = x < v1
            c2 = x < v2
            c3 = x < v3
            v3 = jnp.where(c2, v2, jnp.where(c3, x, v3))
            i3 = jnp.where(c2, i2, jnp.where(c3, ix, i3))
            v2 = jnp.where(c1, v1, jnp.where(c2, x, v2))
            i2 = jnp.where(c1, i1, jnp.where(c2, ix, i2))
            v1 = jnp.where(c1, x, v1)
            i1 = jnp.where(c1, ix, i1)
        cols = []
        vmin = v1
        am = i1
        for m in range(M):
            vmin = jnp.min(v1, axis=1, keepdims=True)
            eqm = v1 == vmin
            am = jnp.min(jnp.where(eqm, i1, big), axis=1, keepdims=True)
            cols.append(am)
            if m < M - 1:
                win = eqm & (i1 == am)
                v1 = jnp.where(win, v2, v1)
                i1 = jnp.where(win, i2, i1)
                v2 = jnp.where(win, v3, v2)
                i2 = jnp.where(win, i3, i2)
                v3 = jnp.where(win, inf, v3)
        nbr = jnp.concatenate(cols, axis=1)
        j_ref[rows, :] = nbr * 2
        acc = jnp.zeros((_CHUNK, 128), jnp.int32)
        for t in range(N_PAD // 128):
            x = d2_ref[rows, pl.ds(t * 128, 128)]
            ix = lane + t * 128
            le = (x < vmin) | ((x == vmin) & (ix <= am))
            acc = acc + le.astype(jnp.int32)
        cnt = jnp.sum(acc, axis=1, keepdims=True)
        bad = jnp.max(jnp.abs(cnt - M)) > 0

        @pl.when(bad)
        def _fallback():
            d = d2_ref[rows, :]
            ci = lax.broadcasted_iota(jnp.int32, (_CHUNK, N_PAD), 1)
            fcols = []
            for m in range(M):
                fvmin = jnp.min(d, axis=1, keepdims=True)
                fam = jnp.min(jnp.where(d == fvmin, ci, N_PAD), axis=1,
                              keepdims=True)
                fcols.append(fam)
                if m < M - 1:
                    d = jnp.where(ci == fam, inf, d)
            j_ref[rows, :] = jnp.concatenate(fcols, axis=1) * 2

        return carry

    lax.fori_loop(0, _RB // _CHUNK, chunk, 0)
    rowid = lax.broadcasted_iota(jnp.int32, (_RB, M), 0)
    i_ref[...] = 2 * (b * _RB + rowid) + 1


def _tc_knn(s2p):
    grid = N_PAD // _RB
    return pl.pallas_call(
        _knn_body,
        grid=(grid,),
        in_specs=[pl.BlockSpec((N_PAD, 2 * COORD_DIM), lambda i: (0, 0))],
        out_specs=[
            pl.BlockSpec((_RB, M), lambda i: (i, 0)),
            pl.BlockSpec((_RB, M), lambda i: (i, 0)),
            pl.BlockSpec((_RB, COORD_DIM), lambda i: (i, 0)),
        ],
        out_shape=[
            jax.ShapeDtypeStruct((N_PAD, M), jnp.int32),
            jax.ShapeDtypeStruct((N_PAD, M), jnp.int32),
            jax.ShapeDtypeStruct((N_PAD, COORD_DIM), jnp.float32),
        ],
        scratch_shapes=[pltpu.VMEM((_RB, N_PAD), jnp.float32)],
    )(s2p)


def kernel(s_l, h):
    h_hor_pad = _sc_gather_feats(h)
    s2 = s_l.reshape(N_HALF, 2 * COORD_DIM)
    s2p = jnp.pad(s2, ((0, N_PAD - N_HALF), (0, 0)))
    j_pad, i_pad, s_hor_pad = _tc_knn(s2p)
    i = i_pad[:N_HALF].reshape(-1)
    j = j_pad[:N_HALF].reshape(-1)
    return (i, j, h_hor_pad[:N_HALF], s_hor_pad[:N_HALF])


# speedup vs baseline: 14.8479x; 1.0144x over previous
import functools

import jax
import jax.numpy as jnp
from jax import lax
from jax.experimental import pallas as pl
from jax.experimental.pallas import tpu as pltpu
from jax.experimental.pallas import tpu_sc as plsc

N_NODES = 10000
N_HALF = 5000
N_PAD = 5120
COORD_DIM = 64
FEAT_DIM = 256
M = 16

_NC = 2
_NS = 16
_NW = _NC * _NS
_ROWS_PER_W = N_PAD // _NW


def _sc_gather_feats(h):
    mesh = plsc.VectorSubcoreMesh(core_axis_name="c", subcore_axis_name="s")

    @functools.partial(
        pl.kernel,
        mesh=mesh,
        out_type=jax.ShapeDtypeStruct((N_PAD, FEAT_DIM), jnp.float32),
        scratch_types=[
            pltpu.VMEM((_ROWS_PER_W,), jnp.int32),
            pltpu.VMEM((_ROWS_PER_W, FEAT_DIM), jnp.float32),
            pltpu.SemaphoreType.DMA,
        ],
    )
    def k(h_hbm, out_hbm, idx_v, buf, sem):
        wid = lax.axis_index("s") * _NC + lax.axis_index("c")
        base = wid * _ROWS_PER_W
        for c in range(_ROWS_PER_W // 16):
            lane = lax.iota(jnp.int32, 16)
            ev = jnp.minimum((base + c * 16 + lane) * 2, N_NODES - 2)
            idx_v[pl.ds(c * 16, 16)] = ev
        pltpu.async_copy(h_hbm.at[idx_v], buf, sem).wait()
        pltpu.sync_copy(buf, out_hbm.at[pl.ds(base, _ROWS_PER_W)])

    return k(h)


_RB = 256
_KT = 512
_NKT = N_PAD // _KT
_CHUNK = 256


def _knn_body(s2_ref, j_ref, i_ref, sh_ref, d2_ref):
    b = pl.program_id(0)
    blk = s2_ref[pl.ds(b * _RB, _RB), :]
    q = blk[:, COORD_DIM:]
    sh_ref[...] = blk[:, :COORD_DIM]
    qs = jnp.sum(q * q, axis=1, keepdims=True)
    for kt in range(_NKT):
        keys_t = s2_ref[pl.ds(kt * _KT, _KT), :COORD_DIM]
        dp = lax.dot_general(
            q, keys_t, dimension_numbers=(((1,), (1,)), ((), ())),
            preferred_element_type=jnp.float32,
        )
        ks_col = jnp.sum(keys_t * keys_t, axis=1, keepdims=True)
        ks_row = lax.transpose(ks_col, (1, 0))
        d2 = qs - 2.0 * dp + ks_row
        if (kt + 1) * _KT > N_HALF:
            col = kt * _KT + lax.broadcasted_iota(jnp.int32, (_RB, _KT), 1)
            d2 = jnp.where(col >= N_HALF, jnp.inf, d2)
        d2_ref[:, pl.ds(kt * _KT, _KT)] = d2

    def chunk(c, carry):
        rows = pl.ds(c * _CHUNK, _CHUNK)
        lane = lax.broadcasted_iota(jnp.int32, (_CHUNK, 128), 1)
        big = jnp.int32(1 << 30)
        inf = jnp.float32(jnp.inf)
        v1 = jnp.full((_CHUNK, 128), inf)
        v2 = jnp.full((_CHUNK, 128), inf)
        v3 = jnp.full((_CHUNK, 128), inf)
        i1 = jnp.full((_CHUNK, 128), big)
        i2 = jnp.full((_CHUNK, 128), big)
        i3 = jnp.full((_CHUNK, 128), big)
        for t in range(N_PAD // 128):
            x = d2_ref[rows, pl.ds(t * 128, 128)]
            ix = lane + t * 128
            c1 = x < v1
            c2 = x < v2
            c3 = x < v3
            v3 = jnp.where(c2, v2, jnp.where(c3, x, v3))
            i3 = jnp.where(c2, i2, jnp.where(c3, ix, i3))
            v2 = jnp.where(c1, v1, jnp.where(c2, x, v2))
            i2 = jnp.where(c1, i1, jnp.where(c2, ix, i2))
            v1 = jnp.where(c1, x, v1)
            i1 = jnp.where(c1, ix, i1)
        cols = []
        vmin = v1
        am = i1
        for m in range(M):
            vmin = jnp.min(v1, axis=1, keepdims=True)
            eqm = v1 == vmin
            am = jnp.min(jnp.where(eqm, i1, big), axis=1, keepdims=True)
            cols.append(am)
            if m < M - 1:
                win = eqm & (i1 == am)
                v1 = jnp.where(win, v2, v1)
                i1 = jnp.where(win, i2, i1)
                v2 = jnp.where(win, v3, v2)
                i2 = jnp.where(win, i3, i2)
                v3 = jnp.where(win, inf, v3)
        nbr = jnp.concatenate(cols, axis=1)
        j_ref[rows, :] = nbr * 2
        acc = jnp.zeros((_CHUNK, 128), jnp.int32)
        for t in range(N_PAD // 128):
            x = d2_ref[rows, pl.ds(t * 128, 128)]
            ix = lane + t * 128
            le = (x < vmin) | ((x == vmin) & (ix <= am))
            acc = acc + le.astype(jnp.int32)
        cnt = jnp.sum(acc, axis=1, keepdims=True)
        bad = jnp.max(jnp.abs(cnt - M)) > 0

        @pl.when(bad)
        def _fallback():
            d = d2_ref[rows, :]
            ci = lax.broadcasted_iota(jnp.int32, (_CHUNK, N_PAD), 1)
            fcols = []
            for m in range(M):
                fvmin = jnp.min(d, axis=1, keepdims=True)
                fam = jnp.min(jnp.where(d == fvmin, ci, N_PAD), axis=1,
                              keepdims=True)
                fcols.append(fam)
                if m < M - 1:
                    d = jnp.where(ci == fam, inf, d)
            j_ref[rows, :] = jnp.concatenate(fcols, axis=1) * 2

        return carry

    lax.fori_loop(0, _RB // _CHUNK, chunk, 0)
    rowid = lax.broadcasted_iota(jnp.int32, (_RB, M), 0)
    i_ref[...] = 2 * (b * _RB + rowid) + 1


def _tc_knn(s2p):
    grid = N_PAD // _RB
    return pl.pallas_call(
        _knn_body,
        grid=(grid,),
        in_specs=[pl.BlockSpec((N_PAD, 2 * COORD_DIM), lambda i: (0, 0))],
        out_specs=[
            pl.BlockSpec((_RB, M), lambda i: (i, 0)),
            pl.BlockSpec((_RB, M), lambda i: (i, 0)),
            pl.BlockSpec((_RB, COORD_DIM), lambda i: (i, 0)),
        ],
        out_shape=[
            jax.ShapeDtypeStruct((N_PAD, M), jnp.int32),
            jax.ShapeDtypeStruct((N_PAD, M), jnp.int32),
            jax.ShapeDtypeStruct((N_PAD, COORD_DIM), jnp.float32),
        ],
        scratch_shapes=[pltpu.VMEM((_RB, N_PAD), jnp.float32)],
    )(s2p)


def kernel(s_l, h):
    h_hor_pad = _sc_gather_feats(h)
    s2 = s_l.reshape(N_HALF, 2 * COORD_DIM)
    s2p = jnp.pad(s2, ((0, N_PAD - N_HALF), (0, 0)))
    j_pad, i_pad, s_hor_pad = _tc_knn(s2p)
    i = i_pad[:N_HALF].reshape(-1)
    j = j_pad[:N_HALF].reshape(-1)
    return (i, j, h_hor_pad[:N_HALF], s_hor_pad[:N_HALF])


# speedup vs baseline: 16.8317x; 1.1336x over previous
import functools

import jax
import jax.numpy as jnp
from jax import lax
from jax.experimental import pallas as pl
from jax.experimental.pallas import tpu as pltpu
from jax.experimental.pallas import tpu_sc as plsc

N_NODES = 10000
N_HALF = 5000
N_PAD = 5120
COORD_DIM = 64
FEAT_DIM = 256
M = 16

_NC = 2
_NS = 16
_NW = _NC * _NS
_ROWS_PER_W = N_PAD // _NW


def _sc_gather_feats(h):
    mesh = plsc.VectorSubcoreMesh(core_axis_name="c", subcore_axis_name="s")

    @functools.partial(
        pl.kernel,
        mesh=mesh,
        out_type=jax.ShapeDtypeStruct((N_PAD, FEAT_DIM), jnp.float32),
        scratch_types=[
            pltpu.VMEM((_ROWS_PER_W,), jnp.int32),
            pltpu.VMEM((_ROWS_PER_W, FEAT_DIM), jnp.float32),
            pltpu.SemaphoreType.DMA,
        ],
    )
    def k(h_hbm, out_hbm, idx_v, buf, sem):
        wid = lax.axis_index("s") * _NC + lax.axis_index("c")
        base = wid * _ROWS_PER_W
        for c in range(_ROWS_PER_W // 16):
            lane = lax.iota(jnp.int32, 16)
            ev = jnp.minimum((base + c * 16 + lane) * 2, N_NODES - 2)
            idx_v[pl.ds(c * 16, 16)] = ev
        pltpu.async_copy(h_hbm.at[idx_v], buf, sem).wait()
        pltpu.sync_copy(buf, out_hbm.at[pl.ds(base, _ROWS_PER_W)])

    return k(h)


_RB = 256
_KT = 512
_NKT = N_PAD // _KT
_CHUNK = 256


def _knn_body(s2_ref, j_ref, i_ref, sh_ref, d2_ref):
    b = pl.program_id(0)
    blk = s2_ref[pl.ds(b * _RB, _RB), :]
    q = blk[:, COORD_DIM:]
    sh_ref[...] = blk[:, :COORD_DIM]
    qs = jnp.sum(q * q, axis=1, keepdims=True)
    for kt in range(_NKT):
        keys_t = s2_ref[pl.ds(kt * _KT, _KT), :COORD_DIM]
        dp = lax.dot_general(
            q, keys_t, dimension_numbers=(((1,), (1,)), ((), ())),
            preferred_element_type=jnp.float32,
        )
        ks_col = jnp.sum(keys_t * keys_t, axis=1, keepdims=True)
        ks_row = lax.transpose(ks_col, (1, 0))
        d2 = qs - 2.0 * dp + ks_row
        if (kt + 1) * _KT > N_HALF:
            col = kt * _KT + lax.broadcasted_iota(jnp.int32, (_RB, _KT), 1)
            d2 = jnp.where(col >= N_HALF, jnp.inf, d2)
        d2_ref[:, pl.ds(kt * _KT, _KT)] = d2

    def chunk(c, carry):
        rows = pl.ds(c * _CHUNK, _CHUNK)
        lane = lax.broadcasted_iota(jnp.int32, (_CHUNK, 128), 1)
        big = jnp.int32(1 << 30)
        inf = jnp.float32(jnp.inf)
        v1 = jnp.full((_CHUNK, 128), inf)
        v2 = jnp.full((_CHUNK, 128), inf)
        v3 = jnp.full((_CHUNK, 128), inf)
        v4 = jnp.full((_CHUNK, 128), inf)
        i1 = jnp.full((_CHUNK, 128), big)
        i2 = jnp.full((_CHUNK, 128), big)
        i3 = jnp.full((_CHUNK, 128), big)
        i4 = jnp.full((_CHUNK, 128), big)
        for t in range(N_PAD // 128):
            x = d2_ref[rows, pl.ds(t * 128, 128)]
            ix = lane + t * 128
            c1 = x < v1
            c2 = x < v2
            c3 = x < v3
            c4 = x < v4
            v4 = jnp.where(c3, v3, jnp.where(c4, x, v4))
            i4 = jnp.where(c3, i3, jnp.where(c4, ix, i4))
            v3 = jnp.where(c2, v2, jnp.where(c3, x, v3))
            i3 = jnp.where(c2, i2, jnp.where(c3, ix, i3))
            v2 = jnp.where(c1, v1, jnp.where(c2, x, v2))
            i2 = jnp.where(c1, i1, jnp.where(c2, ix, i2))
            v1 = jnp.where(c1, x, v1)
            i1 = jnp.where(c1, ix, i1)
        cols = []
        vmin = v1
        am = i1
        for m in range(M):
            vmin = jnp.min(v1, axis=1, keepdims=True)
            eqm = v1 == vmin
            am = jnp.min(jnp.where(eqm, i1, big), axis=1, keepdims=True)
            cols.append(am)
            if m < M - 1:
                win = eqm & (i1 == am)
                v1 = jnp.where(win, v2, v1)
                i1 = jnp.where(win, i2, i1)
                v2 = jnp.where(win, v3, v2)
                i2 = jnp.where(win, i3, i2)
                v3 = jnp.where(win, v4, v3)
                i3 = jnp.where(win, i4, i3)
                v4 = jnp.where(win, inf, v4)
        nbr = jnp.concatenate(cols, axis=1)
        j_ref[rows, :] = nbr * 2
        acc_lt = jnp.zeros((_CHUNK, 128), jnp.int32)
        acc_eq = jnp.zeros((_CHUNK, 128), jnp.int32)
        for t in range(N_PAD // 128):
            x = d2_ref[rows, pl.ds(t * 128, 128)]
            acc_lt = acc_lt + (x < vmin).astype(jnp.int32)
            acc_eq = acc_eq + (x == vmin).astype(jnp.int32)
        cnt_lt = jnp.sum(acc_lt, axis=1, keepdims=True)
        cnt_eq = jnp.sum(acc_eq, axis=1, keepdims=True)
        bad = jnp.max(jnp.abs(cnt_lt - (M - 1)) + jnp.abs(cnt_eq - 1)) > 0

        @pl.when(bad)
        def _fallback():
            d = d2_ref[rows, :]
            ci = lax.broadcasted_iota(jnp.int32, (_CHUNK, N_PAD), 1)
            fcols = []
            for m in range(M):
                fvmin = jnp.min(d, axis=1, keepdims=True)
                fam = jnp.min(jnp.where(d == fvmin, ci, N_PAD), axis=1,
                              keepdims=True)
                fcols.append(fam)
                if m < M - 1:
                    d = jnp.where(ci == fam, inf, d)
            j_ref[rows, :] = jnp.concatenate(fcols, axis=1) * 2

        return carry

    lax.fori_loop(0, _RB // _CHUNK, chunk, 0)
    rowid = lax.broadcasted_iota(jnp.int32, (_RB, M), 0)
    i_ref[...] = 2 * (b * _RB + rowid) + 1


def _tc_knn(s2p):
    grid = N_PAD // _RB
    return pl.pallas_call(
        _knn_body,
        grid=(grid,),
        in_specs=[pl.BlockSpec((N_PAD, 2 * COORD_DIM), lambda i: (0, 0))],
        out_specs=[
            pl.BlockSpec((_RB, M), lambda i: (i, 0)),
            pl.BlockSpec((_RB, M), lambda i: (i, 0)),
            pl.BlockSpec((_RB, COORD_DIM), lambda i: (i, 0)),
        ],
        out_shape=[
            jax.ShapeDtypeStruct((N_PAD, M), jnp.int32),
            jax.ShapeDtypeStruct((N_PAD, M), jnp.int32),
            jax.ShapeDtypeStruct((N_PAD, COORD_DIM), jnp.float32),
        ],
        scratch_shapes=[pltpu.VMEM((_RB, N_PAD), jnp.float32)],
    )(s2p)


def kernel(s_l, h):
    h_hor_pad = _sc_gather_feats(h)
    s2 = s_l.reshape(N_HALF, 2 * COORD_DIM)
    s2p = jnp.pad(s2, ((0, N_PAD - N_HALF), (0, 0)))
    j_pad, i_pad, s_hor_pad = _tc_knn(s2p)
    i = i_pad[:N_HALF].reshape(-1)
    j = j_pad[:N_HALF].reshape(-1)
    return (i, j, h_hor_pad[:N_HALF], s_hor_pad[:N_HALF])
